# speedup vs baseline: 1.0482x; 1.0482x over previous
import functools

import jax
import jax.numpy as jnp
from jax.experimental import pallas as pl
from jax.experimental.pallas import tpu as pltpu

N = 10000
HEADS = 8
C = 16


def _mm_kernel(x_ref, w_ref, o_ref):
    o_ref[...] = jnp.dot(x_ref[...], w_ref[...],
                         preferred_element_type=jnp.float32)


def _matmul(x, w):
    m, k = x.shape
    _, n = w.shape
    bm = 2000
    return pl.pallas_call(
        _mm_kernel,
        grid=(m // bm,),
        in_specs=[pl.BlockSpec((bm, k), lambda i: (i, 0)),
                  pl.BlockSpec((k, n), lambda i: (0, 0))],
        out_specs=pl.BlockSpec((bm, n), lambda i: (i, 0)),
        out_shape=jax.ShapeDtypeStruct((m, n), jnp.float32),
    )(x, w)


def _leaky(v):
    return jnp.where(v > 0, v, 0.2 * v)


def _gat_conv(x, src, dst, W, a_src, a_dst, bias, heads, out_ch, concat):
    n = x.shape[0]
    h = _matmul(x, W).reshape(n, heads, out_ch)
    al_s = jnp.sum(h * a_src[None], axis=-1)
    al_d = jnp.sum(h * a_dst[None], axis=-1)
    maxs = jnp.max(al_s, axis=0)
    m = _leaky(al_d + maxs)
    alpha = _leaky(al_s[src] + al_d[dst])
    w = jnp.exp(alpha - m[dst])
    denom = jax.ops.segment_sum(w, dst, num_segments=n)
    w = w / (denom[dst] + 1e-16)
    out = jax.ops.segment_sum(h[src] * w[:, :, None], dst, num_segments=n)
    out = out.reshape(n, heads * out_ch) if concat else out.mean(axis=1)
    return out + bias


def _bn_elu(h, g, b):
    mu = jnp.mean(h, axis=0)
    v = jnp.mean((h - mu) ** 2, axis=0)
    return jax.nn.elu((h - mu) / jnp.sqrt(v + 1e-5) * g + b)


def kernel(x, edge_index, params):
    p = params
    n = x.shape[0]
    loops = jnp.arange(n, dtype=edge_index.dtype)
    ei = jnp.concatenate([edge_index, jnp.stack([loops, loops])], axis=1)
    src, dst = ei[0], ei[1]
    h = _gat_conv(x, src, dst, p["W0"], p["as0"], p["ad0"], p["b0"], HEADS, C, True)
    h = _bn_elu(h, p["g0"], p["be0"])
    h = _gat_conv(h, src, dst, p["W1"], p["as1"], p["ad1"], p["b1"], HEADS, C, True)
    h = _bn_elu(h, p["g1"], p["be1"])
    h = _gat_conv(h, src, dst, p["W2"], p["as2"], p["ad2"], p["b2"], 1, 128, False)
    h = _bn_elu(h, p["g2"], p["be2"])
    return h


# speedup vs baseline: 6.6372x; 6.3321x over previous
import jax
import jax.numpy as jnp
from jax import lax
from jax.experimental import pallas as pl
from jax.experimental.pallas import tpu as pltpu

N = 10000
NP = 10112
NWORK = 32
BCH = 128
NCHUNK = 81
EPAD = NWORK * BCH * NCHUNK
EROWS = EPAD // 128
ROWS_PER_CHUNK = BCH // 128
ROWS_PER_WORKER = EROWS // NWORK
TILE_ROWS = NP // 16
BM = 2000

_f32 = jnp.float32


def _dense_tail(y, w_ref, acs_ref, acd_ref, h_ref, als_ref, ald_ref, mx_ref, i):
    h = jnp.dot(y, w_ref[...], preferred_element_type=_f32)
    h_ref[...] = h
    als = jnp.dot(h, acs_ref[...], preferred_element_type=_f32)
    ald = jnp.dot(h, acd_ref[...], preferred_element_type=_f32)
    als_ref[...] = als
    ald_ref[...] = ald
    bmax = jnp.broadcast_to(jnp.max(als, axis=0, keepdims=True), (8, 128))

    @pl.when(i == 0)
    def _():
        mx_ref[...] = bmax

    @pl.when(i > 0)
    def _():
        mx_ref[...] = jnp.maximum(mx_ref[...], bmax)


def _entry_kernel(x_ref, w_ref, acs_ref, acd_ref, h_ref, als_ref, ald_ref, mx_ref):
    _dense_tail(x_ref[...], w_ref, acs_ref, acd_ref,
                h_ref, als_ref, ald_ref, mx_ref, pl.program_id(0))


def _entry(x, w, acs, acd):
    return pl.pallas_call(
        _entry_kernel,
        grid=(N // BM,),
        in_specs=[
            pl.BlockSpec((BM, 128), lambda i: (i, 0)),
            pl.BlockSpec((128, 128), lambda i: (0, 0)),
            pl.BlockSpec((128, 128), lambda i: (0, 0)),
            pl.BlockSpec((128, 128), lambda i: (0, 0)),
        ],
        out_specs=[
            pl.BlockSpec((BM, 128), lambda i: (i, 0)),
            pl.BlockSpec((BM, 128), lambda i: (i, 0)),
            pl.BlockSpec((BM, 128), lambda i: (i, 0)),
            pl.BlockSpec((8, 128), lambda i: (0, 0)),
        ],
        out_shape=[
            jax.ShapeDtypeStruct((N, 128), _f32),
            jax.ShapeDtypeStruct((N, 128), _f32),
            jax.ShapeDtypeStruct((N, 128), _f32),
            jax.ShapeDtypeStruct((8, 128), _f32),
        ],
    )(x, w, acs, acd)


def _stats_kernel(op_ref, den_ref, emat_ref, bias_ref, s_ref, sum_ref, sq_ref):
    i = pl.program_id(0)
    o = op_ref[0] + op_ref[1]
    d = den_ref[0] + den_ref[1]
    dexp = jnp.dot(d, emat_ref[...], preferred_element_type=_f32)
    sv = o / (dexp + 1e-16) + bias_ref[...]
    s_ref[...] = sv
    ps = jnp.broadcast_to(jnp.sum(sv, axis=0, keepdims=True), (8, 128))
    pq = jnp.broadcast_to(jnp.sum(sv * sv, axis=0, keepdims=True), (8, 128))

    @pl.when(i == 0)
    def _():
        sum_ref[...] = ps
        sq_ref[...] = pq

    @pl.when(i > 0)
    def _():
        sum_ref[...] = sum_ref[...] + ps
        sq_ref[...] = sq_ref[...] + pq


def _stats(op, den, emat, bias):
    return pl.pallas_call(
        _stats_kernel,
        grid=(N // BM,),
        in_specs=[
            pl.BlockSpec((2, BM, 128), lambda i: (0, i, 0)),
            pl.BlockSpec((2, BM, 16), lambda i: (0, i, 0)),
            pl.BlockSpec((16, 128), lambda i: (0, 0)),
            pl.BlockSpec((1, 128), lambda i: (0, 0)),
        ],
        out_specs=[
            pl.BlockSpec((BM, 128), lambda i: (i, 0)),
            pl.BlockSpec((8, 128), lambda i: (0, 0)),
            pl.BlockSpec((8, 128), lambda i: (0, 0)),
        ],
        out_shape=[
            jax.ShapeDtypeStruct((N, 128), _f32),
            jax.ShapeDtypeStruct((8, 128), _f32),
            jax.ShapeDtypeStruct((8, 128), _f32),
        ],
    )(op, den, emat, bias)


def _bn_elu_block(s_ref, sum_ref, sq_ref, g_ref, be_ref):
    mu = sum_ref[0:1, :] * (1.0 / N)
    var = sq_ref[0:1, :] * (1.0 / N) - mu * mu
    yv = (s_ref[...] - mu) * lax.rsqrt(var + 1e-5) * g_ref[...] + be_ref[...]
    return jnp.where(yv > 0, yv, jnp.exp(yv) - 1.0)


def _apply_next_kernel(s_ref, sum_ref, sq_ref, g_ref, be_ref, w_ref, acs_ref,
                       acd_ref, h_ref, als_ref, ald_ref, mx_ref):
    y = _bn_elu_block(s_ref, sum_ref, sq_ref, g_ref, be_ref)
    _dense_tail(y, w_ref, acs_ref, acd_ref,
                h_ref, als_ref, ald_ref, mx_ref, pl.program_id(0))


def _apply_next(s, ssum, ssq, g, be, w, acs, acd):
    return pl.pallas_call(
        _apply_next_kernel,
        grid=(N // BM,),
        in_specs=[
            pl.BlockSpec((BM, 128), lambda i: (i, 0)),
            pl.BlockSpec((8, 128), lambda i: (0, 0)),
            pl.BlockSpec((8, 128), lambda i: (0, 0)),
            pl.BlockSpec((1, 128), lambda i: (0, 0)),
            pl.BlockSpec((1, 128), lambda i: (0, 0)),
            pl.BlockSpec((128, 128), lambda i: (0, 0)),
            pl.BlockSpec((128, 128), lambda i: (0, 0)),
            pl.BlockSpec((128, 128), lambda i: (0, 0)),
        ],
        out_specs=[
            pl.BlockSpec((BM, 128), lambda i: (i, 0)),
            pl.BlockSpec((BM, 128), lambda i: (i, 0)),
            pl.BlockSpec((BM, 128), lambda i: (i, 0)),
            pl.BlockSpec((8, 128), lambda i: (0, 0)),
        ],
        out_shape=[
            jax.ShapeDtypeStruct((N, 128), _f32),
            jax.ShapeDtypeStruct((N, 128), _f32),
            jax.ShapeDtypeStruct((N, 128), _f32),
            jax.ShapeDtypeStruct((8, 128), _f32),
        ],
    )(s, ssum, ssq, g, be, w, acs, acd)


def _final_kernel(s_ref, sum_ref, sq_ref, g_ref, be_ref, y_ref):
    y_ref[...] = _bn_elu_block(s_ref, sum_ref, sq_ref, g_ref, be_ref)


def _final(s, ssum, ssq, g, be):
    return pl.pallas_call(
        _final_kernel,
        grid=(N // BM,),
        in_specs=[
            pl.BlockSpec((BM, 128), lambda i: (i, 0)),
            pl.BlockSpec((8, 128), lambda i: (0, 0)),
            pl.BlockSpec((8, 128), lambda i: (0, 0)),
            pl.BlockSpec((1, 128), lambda i: (0, 0)),
            pl.BlockSpec((1, 128), lambda i: (0, 0)),
        ],
        out_specs=pl.BlockSpec((BM, 128), lambda i: (i, 0)),
        out_shape=jax.ShapeDtypeStruct((N, 128), _f32),
    )(s, ssum, ssq, g, be)


def kernel(x, edge_index, params):
    p = params

    loops = jnp.arange(N, dtype=jnp.int32)
    src = jnp.concatenate([edge_index[0].astype(jnp.int32), loops])
    dst = jnp.concatenate([edge_index[1].astype(jnp.int32), loops])
    e_real = src.shape[0]
    src = jnp.pad(src, (0, max(0, EPAD - e_real)))[:EPAD].reshape(EROWS, 128)
    dst = jnp.pad(dst, (0, max(0, EPAD - e_real)),
                  constant_values=N)[:EPAD].reshape(EROWS, 128)

    rr = jnp.pad(jnp.repeat(jnp.eye(8, dtype=_f32), 16, axis=0),
                 ((0, 0), (0, 120)))
    acs0 = p["as0"].reshape(128)[:, None] * rr
    acd0 = p["ad0"].reshape(128)[:, None] * rr
    acs1 = p["as1"].reshape(128)[:, None] * rr
    acd1 = p["ad1"].reshape(128)[:, None] * rr
    m8 = (jnp.arange(128) < 8).astype(_f32)
    acs2 = p["as2"].reshape(128)[:, None] * m8[None, :]
    acd2 = p["ad2"].reshape(128)[:, None] * m8[None, :]
    emat8 = jnp.concatenate(
        [jnp.repeat(jnp.eye(8, dtype=_f32), 16, axis=1),
         jnp.zeros((8, 128), _f32)], axis=0)
    emat1 = jnp.zeros((16, 128), _f32).at[0].set(1.0)

    def edge_phase(h, als, ald, mx):
        sv = src.reshape(-1)
        dv = dst.reshape(-1)
        ald_p = jnp.pad(ald, ((0, NP - N), (0, 0)))
        gs = als[sv][:, :16]
        gd = ald_p[dv][:, :16]
        t = gs + gd
        t = jnp.where(t > 0, t, 0.2 * t)
        m = gd + mx[0, :16][None, :]
        m = jnp.where(m > 0, m, 0.2 * m)
        w = jnp.exp(t - m)
        w = jnp.where(jnp.arange(16)[None, :] < 8, w, 0.0)
        hw = h[sv] * jnp.repeat(w[:, :8], 16, axis=1)
        oacc = jnp.zeros((NP, 128), _f32).at[dv].add(hw)
        dacc = jnp.zeros((NP, 16), _f32).at[dv].add(w)
        return (jnp.stack([oacc, jnp.zeros_like(oacc)]),
                jnp.stack([dacc, jnp.zeros_like(dacc)]))

    h, als, ald, mx = _entry(x, p["W0"], acs0, acd0)
    op, dp = edge_phase(h, als, ald, mx)
    s, ssum, ssq = _stats(op, dp, emat8, p["b0"].reshape(1, 128))
    h, als, ald, mx = _apply_next(s, ssum, ssq, p["g0"].reshape(1, 128),
                                  p["be0"].reshape(1, 128), p["W1"], acs1, acd1)
    op, dp = edge_phase(h, als, ald, mx)
    s, ssum, ssq = _stats(op, dp, emat8, p["b1"].reshape(1, 128))
    h, als, ald, mx = _apply_next(s, ssum, ssq, p["g1"].reshape(1, 128),
                                  p["be1"].reshape(1, 128), p["W2"], acs2, acd2)
    op, dp = edge_phase(h, als, ald, mx)
    s, ssum, ssq = _stats(op, dp, emat1, p["b2"].reshape(1, 128))
    return _final(s, ssum, ssq, p["g2"].reshape(1, 128), p["be2"].reshape(1, 128))
